# baseline (speedup 1.0000x reference)
import jax
import jax.numpy as jnp
import numpy as np
from jax.experimental import pallas as pl

_FORMULA_OH_SIZES = [20, 20, 20, 20, 20]
_SPECT_BIN_N = 512


def _dot_t(x, w):
    return jax.lax.dot_general(
        x, w, (((1,), (1,)), ((), ())), preferred_element_type=jnp.float32)


def _ln(x, g, b, eps=1e-5):
    m = jnp.mean(x, axis=-1, keepdims=True)
    v = jnp.mean((x - m) ** 2, axis=-1, keepdims=True)
    return (x - m) * jax.lax.rsqrt(v + eps) * g + b


def _full_kernel(
    x_ref,
    mask_ref,
    elem_ref,
    subs_ref,
    mass_ref,
    inten_ref,
    ln_g_ref, ln_b_ref,
    wih_ref,
    whh_ref,
    bih_ref, bhh_ref,
    l1w_ref,
    l1b_ref,
    l2aw_ref, l2ab_ref,
    l2bw_ref, l2bb_ref,
    ln2g_ref, ln2b_ref,
    sw_ref,
    sb_ref,
    spect_ref,
    probs_ref,
):
    B, S, A = subs_ref.shape
    G = x_ref.shape[2]
    M = mass_ref.shape[2]
    E = elem_ref.shape[2]
    F = 20 * E

    col = jax.lax.broadcasted_iota(jnp.int32, (E, F), 1)
    rowi = jax.lax.broadcasted_iota(jnp.int32, (E, F), 0)
    seg = (col // 20 == rowi).astype(jnp.float32)
    colmod = (jax.lax.broadcasted_iota(jnp.int32, (S, F), 1) % 20).astype(jnp.float32)

    h_rows = []
    pf_rows = []
    for b in range(B):
        x = x_ref[b]
        mask = mask_ref[b]
        subs_raw = subs_ref[b]
        subs_f = subs_raw * mask

        masked_x = x * mask.reshape(A, 1)
        swvs = jnp.dot(subs_f, masked_x, preferred_element_type=jnp.float32)
        size = jnp.sum(subs_f, axis=1, keepdims=True) + 0.0001
        h_rows.append(_ln(swvs / size, ln_g_ref[...], ln_b_ref[...]))

        p_mat = jnp.dot(elem_ref[b], seg, preferred_element_type=jnp.float32)
        cx = jnp.dot(subs_raw, p_mat, preferred_element_type=jnp.float32)
        thresh = jnp.clip(cx, 0.0, 19.0)
        pf_rows.append((colmod >= thresh).astype(jnp.float32))

    h = jnp.concatenate(h_rows, axis=0)
    pf = jnp.concatenate(pf_rows, axis=0)

    gi = _dot_t(pf, wih_ref[...]) + bih_ref[...]
    gh = _dot_t(h, whh_ref[...]) + bhh_ref[...]
    i_r, i_z, i_n = gi[:, :G], gi[:, G:2 * G], gi[:, 2 * G:]
    h_r, h_z, h_n = gh[:, :G], gh[:, G:2 * G], gh[:, 2 * G:]
    r = jax.nn.sigmoid(i_r + h_r)
    z = jax.nn.sigmoid(i_z + h_z)
    n = jnp.tanh(i_n + r * h_n)
    hn = (1.0 - z) * n + z * h

    x1 = jax.nn.relu(_dot_t(hn, l1w_ref[...]) + l1b_ref[...])
    x2 = jax.nn.relu(_dot_t(x1, l2aw_ref[...]) + l2ab_ref[...])
    x2 = jax.nn.relu(_dot_t(x2, l2bw_ref[...]) + l2bb_ref[...])

    gw = ln2g_ref[...] * sw_ref[...]
    c2 = jnp.sum(ln2b_ref[...] * sw_ref[...]) + sb_ref[0, 0]
    mu = jnp.mean(x2, axis=1, keepdims=True)
    d = x2 - mu
    v = jnp.mean(d * d, axis=1, keepdims=True)
    sgw = jnp.sum(d * gw, axis=1, keepdims=True)
    scores = jax.lax.rsqrt(v + 1e-5) * sgw + c2

    lane32 = jax.lax.broadcasted_iota(jnp.int32, (S, 32), 1).astype(jnp.float32)
    lane16 = jax.lax.broadcasted_iota(jnp.int32, (S, 16), 1).astype(jnp.float32)
    for b in range(B):
        sc = scores[b * S:(b + 1) * S]
        smax = jnp.max(sc, axis=0, keepdims=True)
        e = jnp.exp(sc - smax)
        probs = e / jnp.sum(e, axis=0, keepdims=True)
        probs_ref[b] = probs

        bins = jnp.clip(jnp.round(mass_ref[b]), 0.0, float(_SPECT_BIN_N - 1))
        coarse = jnp.floor(bins * 0.0625)
        fine = bins - 16.0 * coarse
        contrib = inten_ref[b] * probs
        wc_parts = []
        f_parts = []
        for m in range(M):
            ohc = (coarse[:, m:m + 1] == lane32).astype(jnp.float32)
            wc_parts.append(contrib[:, m:m + 1] * ohc)
            f_parts.append((fine[:, m:m + 1] == lane16).astype(jnp.float32))
        wc = jnp.concatenate(wc_parts, axis=0)
        fh = jnp.concatenate(f_parts, axis=0)
        spect_ref[b] = jax.lax.dot_general(
            wc, fh, (((0,), (0,)), ((), ())),
            preferred_element_type=jnp.float32)


def kernel(vert_feat_in, vert_mask_in, vert_element_oh, adj_oh, atom_subsets,
           atom_subsets_peaks, ln_g, ln_b, gru_w_ih, gru_w_hh, gru_b_ih,
           gru_b_hh, l1_w, l1_b, l2a_w, l2a_b, l2b_w, l2b_b, ln2_g, ln2_b,
           score_w, score_b):
    B, A, GF0, HW = vert_feat_in.shape
    G = GF0 * HW
    S = atom_subsets.shape[1]
    M = atom_subsets_peaks.shape[2]
    E = vert_element_oh.shape[2]
    F = int(np.sum(_FORMULA_OH_SIZES))
    D = l1_w.shape[0]

    x = vert_feat_in.reshape(B, A, G)
    mask3 = vert_mask_in.reshape(B, 1, A)
    elem_f = vert_element_oh.astype(jnp.float32)
    subs_f = atom_subsets.astype(jnp.float32)
    mass = atom_subsets_peaks[..., 0]
    inten = atom_subsets_peaks[..., 1]

    row = lambda v: v.reshape(1, -1)

    spect3, probs3 = pl.pallas_call(
        _full_kernel,
        out_shape=[
            jax.ShapeDtypeStruct((B, 32, 16), jnp.float32),
            jax.ShapeDtypeStruct((B, S, 1), jnp.float32),
        ],
    )(
        x, mask3, elem_f, subs_f, mass, inten,
        row(ln_g), row(ln_b),
        gru_w_ih, gru_w_hh, row(gru_b_ih), row(gru_b_hh),
        l1_w, row(l1_b), l2a_w, row(l2a_b), l2b_w, row(l2b_b),
        row(ln2_g), row(ln2_b), score_w, score_b.reshape(1, 1),
    )
    return spect3.reshape(B, _SPECT_BIN_N), probs3.reshape(B, S)


# speedup vs baseline: 1.0691x; 1.0691x over previous
import jax
import jax.numpy as jnp
import numpy as np
from jax.experimental import pallas as pl

_FORMULA_OH_SIZES = [20, 20, 20, 20, 20]
_SPECT_BIN_N = 512


def _dot_t(x, w):
    return jax.lax.dot_general(
        x, w, (((1,), (1,)), ((), ())), preferred_element_type=jnp.float32)


def _ln(x, g, b, eps=1e-5):
    m = jnp.mean(x, axis=-1, keepdims=True)
    v = jnp.mean((x - m) ** 2, axis=-1, keepdims=True)
    return (x - m) * jax.lax.rsqrt(v + eps) * g + b


def _full_kernel(
    x_ref,
    mask_ref,
    elem_ref,
    subs_ref,
    mass_ref,
    inten_ref,
    ln_g_ref, ln_b_ref,
    wih_ref,
    whh_ref,
    bih_ref, bhh_ref,
    l1w_ref,
    l1b_ref,
    l2aw_ref, l2ab_ref,
    l2bw_ref, l2bb_ref,
    ln2g_ref, ln2b_ref,
    sw_ref,
    sb_ref,
    spect_ref,
    probs_ref,
):
    B, S, A = subs_ref.shape
    G = x_ref.shape[2]
    M = mass_ref.shape[2]
    E = elem_ref.shape[2]
    F = 20 * E

    col = jax.lax.broadcasted_iota(jnp.int32, (E, F), 1)
    rowi = jax.lax.broadcasted_iota(jnp.int32, (E, F), 0)
    seg = (col // 20 == rowi).astype(jnp.float32)
    colmod = (jax.lax.broadcasted_iota(jnp.int32, (S, F), 1) % 20).astype(jnp.float32)

    h_rows = []
    pf_rows = []
    for b in range(B):
        x = x_ref[b]
        mask = mask_ref[b]
        subs_raw = subs_ref[b]
        subs_f = subs_raw * mask

        masked_x = x * mask.reshape(A, 1)
        swvs = jnp.dot(subs_f, masked_x, preferred_element_type=jnp.float32)
        size = jnp.sum(subs_f, axis=1, keepdims=True) + 0.0001
        h_rows.append(_ln(swvs / size, ln_g_ref[...], ln_b_ref[...]))

        p_mat = jnp.dot(elem_ref[b], seg, preferred_element_type=jnp.float32)
        cx = jnp.dot(subs_raw, p_mat, preferred_element_type=jnp.float32)
        thresh = jnp.clip(cx, 0.0, 19.0)
        pf_rows.append((colmod >= thresh).astype(jnp.float32))

    h = jnp.concatenate(h_rows, axis=0)
    pf = jnp.concatenate(pf_rows, axis=0)

    gi = _dot_t(pf, wih_ref[...]) + bih_ref[...]
    gh = _dot_t(h, whh_ref[...]) + bhh_ref[...]
    i_r, i_z, i_n = gi[:, :G], gi[:, G:2 * G], gi[:, 2 * G:]
    h_r, h_z, h_n = gh[:, :G], gh[:, G:2 * G], gh[:, 2 * G:]
    r = jax.nn.sigmoid(i_r + h_r)
    z = jax.nn.sigmoid(i_z + h_z)
    n = jnp.tanh(i_n + r * h_n)
    hn = (1.0 - z) * n + z * h

    x1 = jax.nn.relu(_dot_t(hn, l1w_ref[...]) + l1b_ref[...])
    x2 = jax.nn.relu(_dot_t(x1, l2aw_ref[...]) + l2ab_ref[...])
    x2 = jax.nn.relu(_dot_t(x2, l2bw_ref[...]) + l2bb_ref[...])

    gw = ln2g_ref[...] * sw_ref[...]
    c2 = jnp.sum(ln2b_ref[...] * sw_ref[...]) + sb_ref[0, 0]
    mu = jnp.mean(x2, axis=1, keepdims=True)
    d = x2 - mu
    v = jnp.mean(d * d, axis=1, keepdims=True)
    sgw = jnp.sum(d * gw, axis=1, keepdims=True)
    scores = jax.lax.rsqrt(v + 1e-5) * sgw + c2

    lane = jax.lax.broadcasted_iota(jnp.int32, (S, _SPECT_BIN_N), 1).astype(jnp.float32)
    for b in range(B):
        sc = scores[b * S:(b + 1) * S]
        smax = jnp.max(sc, axis=0, keepdims=True)
        e = jnp.exp(sc - smax)
        probs = e / jnp.sum(e, axis=0, keepdims=True)
        probs_ref[b] = probs

        bins = jnp.clip(jnp.round(mass_ref[b]), 0.0, float(_SPECT_BIN_N - 1))
        contrib = inten_ref[b] * probs
        acc = jnp.zeros((S, _SPECT_BIN_N), jnp.float32)
        for m in range(M):
            onehot = (bins[:, m:m + 1] == lane).astype(jnp.float32)
            acc = acc + contrib[:, m:m + 1] * onehot
        spect_ref[b] = jnp.sum(acc, axis=0, keepdims=True)


def kernel(vert_feat_in, vert_mask_in, vert_element_oh, adj_oh, atom_subsets,
           atom_subsets_peaks, ln_g, ln_b, gru_w_ih, gru_w_hh, gru_b_ih,
           gru_b_hh, l1_w, l1_b, l2a_w, l2a_b, l2b_w, l2b_b, ln2_g, ln2_b,
           score_w, score_b):
    B, A, GF0, HW = vert_feat_in.shape
    G = GF0 * HW
    S = atom_subsets.shape[1]
    M = atom_subsets_peaks.shape[2]
    E = vert_element_oh.shape[2]
    F = int(np.sum(_FORMULA_OH_SIZES))
    D = l1_w.shape[0]

    x = vert_feat_in.reshape(B, A, G)
    mask3 = vert_mask_in.reshape(B, 1, A)
    elem_f = vert_element_oh.astype(jnp.float32)
    subs_f = atom_subsets.astype(jnp.float32)
    mass = atom_subsets_peaks[..., 0]
    inten = atom_subsets_peaks[..., 1]

    row = lambda v: v.reshape(1, -1)

    spect3, probs3 = pl.pallas_call(
        _full_kernel,
        out_shape=[
            jax.ShapeDtypeStruct((B, 1, _SPECT_BIN_N), jnp.float32),
            jax.ShapeDtypeStruct((B, S, 1), jnp.float32),
        ],
    )(
        x, mask3, elem_f, subs_f, mass, inten,
        row(ln_g), row(ln_b),
        gru_w_ih, gru_w_hh, row(gru_b_ih), row(gru_b_hh),
        l1_w, row(l1_b), l2a_w, row(l2a_b), l2b_w, row(l2b_b),
        row(ln2_g), row(ln2_b), score_w, score_b.reshape(1, 1),
    )
    return spect3.reshape(B, _SPECT_BIN_N), probs3.reshape(B, S)


# speedup vs baseline: 1.1564x; 1.0816x over previous
import jax
import jax.numpy as jnp
import numpy as np
from jax.experimental import pallas as pl

_FORMULA_OH_SIZES = [20, 20, 20, 20, 20]
_SPECT_BIN_N = 512


def _dot_t(x, w):
    return jax.lax.dot_general(
        x, w, (((1,), (1,)), ((), ())), preferred_element_type=jnp.float32)


def _ln(x, g, b, eps=1e-5):
    m = jnp.mean(x, axis=-1, keepdims=True)
    v = jnp.mean((x - m) ** 2, axis=-1, keepdims=True)
    return (x - m) * jax.lax.rsqrt(v + eps) * g + b


def _full_kernel(
    x_ref,
    mask_ref,
    elem_ref,
    subs_ref,
    peaks_ref,
    ln_g_ref, ln_b_ref,
    wih_ref,
    whh_ref,
    bih_ref, bhh_ref,
    l1w_ref,
    l1b_ref,
    l2aw_ref, l2ab_ref,
    l2bw_ref, l2bb_ref,
    ln2g_ref, ln2b_ref,
    sw_ref,
    sb_ref,
    spect_ref,
    probs_ref,
):
    B, S, A = subs_ref.shape
    G = x_ref.shape[2]
    M2 = peaks_ref.shape[2]
    E = elem_ref.shape[2]
    F = 20 * E

    col = jax.lax.broadcasted_iota(jnp.int32, (E, F), 1)
    rowi = jax.lax.broadcasted_iota(jnp.int32, (E, F), 0)
    seg = (col // 20 == rowi).astype(jnp.float32)
    colmod = (jax.lax.broadcasted_iota(jnp.int32, (S, F), 1) % 20).astype(jnp.float32)

    h_rows = []
    pf_rows = []
    for b in range(B):
        x = x_ref[b]
        mask = mask_ref[b]
        subs_raw = subs_ref[b].astype(jnp.float32)
        subs_f = subs_raw * mask

        masked_x = x * mask.reshape(A, 1)
        swvs = jnp.dot(subs_f, masked_x, preferred_element_type=jnp.float32)
        size = jnp.sum(subs_f, axis=1, keepdims=True) + 0.0001
        h_rows.append(_ln(swvs / size, ln_g_ref[...], ln_b_ref[...]))

        p_mat = jnp.dot(elem_ref[b].astype(jnp.float32), seg,
                        preferred_element_type=jnp.float32)
        cx = jnp.dot(subs_raw, p_mat, preferred_element_type=jnp.float32)
        thresh = jnp.clip(cx, 0.0, 19.0)
        pf_rows.append((colmod >= thresh).astype(jnp.float32))

    h = jnp.concatenate(h_rows, axis=0)
    pf = jnp.concatenate(pf_rows, axis=0)

    gi = _dot_t(pf, wih_ref[...]) + bih_ref[...]
    gh = _dot_t(h, whh_ref[...]) + bhh_ref[...]
    i_r, i_z, i_n = gi[:, :G], gi[:, G:2 * G], gi[:, 2 * G:]
    h_r, h_z, h_n = gh[:, :G], gh[:, G:2 * G], gh[:, 2 * G:]
    r = jax.nn.sigmoid(i_r + h_r)
    z = jax.nn.sigmoid(i_z + h_z)
    n = jnp.tanh(i_n + r * h_n)
    hn = (1.0 - z) * n + z * h

    x1 = jax.nn.relu(_dot_t(hn, l1w_ref[...]) + l1b_ref[...])
    x2 = jax.nn.relu(_dot_t(x1, l2aw_ref[...]) + l2ab_ref[...])
    x2 = jax.nn.relu(_dot_t(x2, l2bw_ref[...]) + l2bb_ref[...])

    gw = ln2g_ref[...] * sw_ref[...]
    c2 = jnp.sum(ln2b_ref[...] * sw_ref[...]) + sb_ref[0]
    mu = jnp.mean(x2, axis=1, keepdims=True)
    d = x2 - mu
    v = jnp.mean(d * d, axis=1, keepdims=True)
    sgw = jnp.sum(d * gw, axis=1, keepdims=True)
    scores = jax.lax.rsqrt(v + 1e-5) * sgw + c2

    lane = jax.lax.broadcasted_iota(jnp.int32, (S, _SPECT_BIN_N), 1).astype(jnp.float32)
    for b in range(B):
        sc = scores[b * S:(b + 1) * S]
        smax = jnp.max(sc, axis=0, keepdims=True)
        e = jnp.exp(sc - smax)
        probs = e / jnp.sum(e, axis=0, keepdims=True)
        probs_ref[b] = probs

        peaks = peaks_ref[b]
        bins = jnp.clip(jnp.round(peaks), 0.0, float(_SPECT_BIN_N - 1))
        contrib = peaks * probs
        acc = jnp.zeros((S, _SPECT_BIN_N), jnp.float32)
        for m in range(M2 // 2):
            onehot = (bins[:, 2 * m:2 * m + 1] == lane).astype(jnp.float32)
            acc = acc + contrib[:, 2 * m + 1:2 * m + 2] * onehot
        spect_ref[b] = jnp.sum(acc, axis=0, keepdims=True)


def kernel(vert_feat_in, vert_mask_in, vert_element_oh, adj_oh, atom_subsets,
           atom_subsets_peaks, ln_g, ln_b, gru_w_ih, gru_w_hh, gru_b_ih,
           gru_b_hh, l1_w, l1_b, l2a_w, l2a_b, l2b_w, l2b_b, ln2_g, ln2_b,
           score_w, score_b):
    B, A, GF0, HW = vert_feat_in.shape
    G = GF0 * HW
    S = atom_subsets.shape[1]
    M = atom_subsets_peaks.shape[2]

    x = vert_feat_in.reshape(B, A, G)
    mask3 = vert_mask_in.reshape(B, 1, A)
    peaks2 = atom_subsets_peaks.reshape(B, S, 2 * M)

    spect3, probs3 = pl.pallas_call(
        _full_kernel,
        out_shape=[
            jax.ShapeDtypeStruct((B, 1, _SPECT_BIN_N), jnp.float32),
            jax.ShapeDtypeStruct((B, S, 1), jnp.float32),
        ],
    )(
        x, mask3, vert_element_oh, atom_subsets, peaks2,
        ln_g, ln_b,
        gru_w_ih, gru_w_hh, gru_b_ih, gru_b_hh,
        l1_w, l1_b, l2a_w, l2a_b, l2b_w, l2b_b,
        ln2_g, ln2_b, score_w, score_b,
    )
    return spect3.reshape(B, _SPECT_BIN_N), probs3.reshape(B, S)


# speedup vs baseline: 1.3109x; 1.1336x over previous
import jax
import jax.numpy as jnp
import numpy as np
from jax.experimental import pallas as pl

_FORMULA_OH_SIZES = [20, 20, 20, 20, 20]
_SPECT_BIN_N = 512


def _dot_t(x, w):
    return jax.lax.dot_general(
        x, w, (((1,), (1,)), ((), ())), preferred_element_type=jnp.float32)


def _dot_tl(xt, y):
    return jax.lax.dot_general(
        xt, y, (((0,), (0,)), ((), ())), preferred_element_type=jnp.float32)


def _ln(x, g, b, eps=1e-5):
    m = jnp.mean(x, axis=-1, keepdims=True)
    v = jnp.mean((x - m) ** 2, axis=-1, keepdims=True)
    return (x - m) * jax.lax.rsqrt(v + eps) * g + b


def _full_kernel(
    x_ref,
    mask_ref,
    elem_ref,
    subs_ref,
    peaks_ref,
    ln_g_ref, ln_b_ref,
    wih_ref,
    whh_ref,
    bih_ref, bhh_ref,
    l1w_ref,
    l1b_ref,
    l2aw_ref, l2ab_ref,
    l2bw_ref, l2bb_ref,
    ln2g_ref, ln2b_ref,
    sw_ref,
    sb_ref,
    spect_ref,
    probs_ref,
):
    B, A, S = subs_ref.shape
    G = x_ref.shape[2]
    M2 = peaks_ref.shape[1]
    E = elem_ref.shape[0]
    F = 20 * E
    HW = 4
    GF0 = G // HW

    rowp = jax.lax.broadcasted_iota(jnp.int32, (G, G), 0)
    colp = jax.lax.broadcasted_iota(jnp.int32, (G, G), 1)
    perm = ((rowp % GF0) * HW + rowp // GF0 == colp).astype(jnp.float32)
    x_all = jnp.dot(x_ref[...].reshape(B * A, G), perm,
                    preferred_element_type=jnp.float32)

    col = jax.lax.broadcasted_iota(jnp.int32, (E, F), 1)
    rowi = jax.lax.broadcasted_iota(jnp.int32, (E, F), 0)
    seg = (col // 20 == rowi).astype(jnp.float32)
    p_all = _dot_tl(elem_ref[...].astype(jnp.float32), seg)
    colmod = (jax.lax.broadcasted_iota(jnp.int32, (S, F), 1) % 20).astype(jnp.float32)

    ones_a = jnp.ones((A, 1), jnp.float32)

    h_rows = []
    pf_rows = []
    for b in range(B):
        subs_t = subs_ref[b].astype(jnp.float32)
        mask_t = mask_ref[b].reshape(A, 1)
        subs_m = subs_t * mask_t
        subs_mm = subs_m * mask_t

        x_b = x_all[b * A:(b + 1) * A]
        swvs = _dot_tl(subs_mm, x_b)
        size = _dot_tl(subs_m, ones_a) + 0.0001
        h_rows.append(_ln(swvs / size, ln_g_ref[...], ln_b_ref[...]))

        cx = _dot_tl(subs_t, p_all[b * A:(b + 1) * A])
        thresh = jnp.clip(cx, 0.0, 19.0)
        pf_rows.append((colmod >= thresh).astype(jnp.float32))

    h = jnp.concatenate(h_rows, axis=0)
    pf = jnp.concatenate(pf_rows, axis=0)

    gi = jnp.dot(pf, wih_ref[...], preferred_element_type=jnp.float32) + bih_ref[...]
    gh = _dot_t(h, whh_ref[...]) + bhh_ref[...]
    i_r, i_z, i_n = gi[:, :G], gi[:, G:2 * G], gi[:, 2 * G:]
    h_r, h_z, h_n = gh[:, :G], gh[:, G:2 * G], gh[:, 2 * G:]
    r = jax.nn.sigmoid(i_r + h_r)
    z = jax.nn.sigmoid(i_z + h_z)
    n = jnp.tanh(i_n + r * h_n)
    hn = (1.0 - z) * n + z * h

    x1 = jax.nn.relu(_dot_t(hn, l1w_ref[...]) + l1b_ref[...])
    x2 = jax.nn.relu(_dot_t(x1, l2aw_ref[...]) + l2ab_ref[...])
    x2 = jax.nn.relu(_dot_t(x2, l2bw_ref[...]) + l2bb_ref[...])

    gw = ln2g_ref[...] * sw_ref[...]
    c2 = jnp.sum(ln2b_ref[...] * sw_ref[...]) + sb_ref[0]
    mu = jnp.mean(x2, axis=1, keepdims=True)
    d = x2 - mu
    v = jnp.mean(d * d, axis=1, keepdims=True)
    sgw = jnp.sum(d * gw, axis=1, keepdims=True)
    scores = jax.lax.rsqrt(v + 1e-5) * sgw + c2

    lane = jax.lax.broadcasted_iota(jnp.int32, (S, _SPECT_BIN_N), 1).astype(jnp.float32)
    for b in range(B):
        sc = scores[b * S:(b + 1) * S]
        smax = jnp.max(sc, axis=0, keepdims=True)
        e = jnp.exp(sc - smax)
        probs = e / jnp.sum(e, axis=0, keepdims=True)
        probs_ref[b] = jnp.transpose(probs, (1, 0))[0]

        peaks = jnp.transpose(peaks_ref[b], (1, 0))
        bins = jnp.clip(jnp.round(peaks), 0.0, float(_SPECT_BIN_N - 1))
        contrib = peaks * probs
        acc = jnp.zeros((S, _SPECT_BIN_N), jnp.float32)
        for m in range(M2 // 2):
            onehot = (bins[:, 2 * m:2 * m + 1] == lane).astype(jnp.float32)
            acc = acc + contrib[:, 2 * m + 1:2 * m + 2] * onehot
        spect_ref[b] = jnp.sum(acc, axis=0)


def kernel(vert_feat_in, vert_mask_in, vert_element_oh, adj_oh, atom_subsets,
           atom_subsets_peaks, ln_g, ln_b, gru_w_ih, gru_w_hh, gru_b_ih,
           gru_b_hh, l1_w, l1_b, l2a_w, l2a_b, l2b_w, l2b_b, ln2_g, ln2_b,
           score_w, score_b):
    B, A, GF0, HW = vert_feat_in.shape
    G = GF0 * HW
    S = atom_subsets.shape[1]
    M = atom_subsets_peaks.shape[2]
    E = vert_element_oh.shape[2]

    x_v = vert_feat_in.transpose(0, 1, 3, 2).reshape(B, A, G)
    mask3 = vert_mask_in.reshape(B, 1, A)
    elem_v = vert_element_oh.transpose(2, 0, 1).reshape(E, B * A)
    subs_v = atom_subsets.transpose(0, 2, 1)
    peaks_v = atom_subsets_peaks.transpose(0, 2, 3, 1).reshape(B, 2 * M, S)
    wih_v = gru_w_ih.T

    spect, probs = pl.pallas_call(
        _full_kernel,
        out_shape=[
            jax.ShapeDtypeStruct((B, _SPECT_BIN_N), jnp.float32),
            jax.ShapeDtypeStruct((B, S), jnp.float32),
        ],
    )(
        x_v, mask3, elem_v, subs_v, peaks_v,
        ln_g, ln_b,
        wih_v, gru_w_hh, gru_b_ih, gru_b_hh,
        l1_w, l1_b, l2a_w, l2a_b, l2b_w, l2b_b,
        ln2_g, ln2_b, score_w, score_b,
    )
    return spect, probs


# speedup vs baseline: 2.0925x; 1.5962x over previous
import jax
import jax.numpy as jnp
import numpy as np
from jax.experimental import pallas as pl

_FORMULA_OH_SIZES = [20, 20, 20, 20, 20]
_SPECT_BIN_N = 512


def _dot_t(x, w):
    return jax.lax.dot_general(
        x, w, (((1,), (1,)), ((), ())), preferred_element_type=jnp.float32)


def _dot_tl(xt, y):
    return jax.lax.dot_general(
        xt, y, (((0,), (0,)), ((), ())), preferred_element_type=jnp.float32)


def _ln(x, g, b, eps=1e-5):
    m = jnp.mean(x, axis=-1, keepdims=True)
    v = jnp.mean((x - m) ** 2, axis=-1, keepdims=True)
    return (x - m) * jax.lax.rsqrt(v + eps) * g + b


def _full_kernel(
    x_ref,
    mask_ref,
    elem_ref,
    subs_ref,
    peaks_ref,
    ln_g_ref, ln_b_ref,
    wih_ref,
    whh_ref,
    bih_ref, bhh_ref,
    l1w_ref,
    l1b_ref,
    l2aw_ref, l2ab_ref,
    l2bw_ref, l2bb_ref,
    ln2g_ref, ln2b_ref,
    sw_ref,
    sb_ref,
    spect_ref,
    probs_ref,
):
    B, A, S = subs_ref.shape
    G = x_ref.shape[2]
    M2 = peaks_ref.shape[1]
    E = elem_ref.shape[0]
    F = 20 * E
    HW = 4
    GF0 = G // HW

    rowp = jax.lax.broadcasted_iota(jnp.int32, (G, G), 0)
    colp = jax.lax.broadcasted_iota(jnp.int32, (G, G), 1)
    perm = ((rowp % GF0) * HW + rowp // GF0 == colp).astype(jnp.float32)
    x_all = jnp.dot(x_ref[...].reshape(B * A, G), perm,
                    preferred_element_type=jnp.float32)

    col = jax.lax.broadcasted_iota(jnp.int32, (E, F), 1)
    rowi = jax.lax.broadcasted_iota(jnp.int32, (E, F), 0)
    seg = (col // 20 == rowi).astype(jnp.float32)
    p_all = _dot_tl(elem_ref[...].astype(jnp.float32), seg)
    colmod = (jax.lax.broadcasted_iota(jnp.int32, (S, F), 1) % 20).astype(jnp.float32)

    ones_a = jnp.ones((A, 1), jnp.float32)

    h_rows = []
    pf_rows = []
    for b in range(B):
        subs_t = subs_ref[b].astype(jnp.float32)
        mask_t = mask_ref[b].reshape(A, 1)
        subs_m = subs_t * mask_t
        subs_mm = subs_m * mask_t

        x_b = x_all[b * A:(b + 1) * A]
        swvs = _dot_tl(subs_mm, x_b)
        size = _dot_tl(subs_m, ones_a) + 0.0001
        h_rows.append(_ln(swvs / size, ln_g_ref[...], ln_b_ref[...]))

        cx = _dot_tl(subs_t, p_all[b * A:(b + 1) * A])
        thresh = jnp.clip(cx, 0.0, 19.0)
        pf_rows.append((colmod >= thresh).astype(jnp.float32))

    h = jnp.concatenate(h_rows, axis=0)
    pf = jnp.concatenate(pf_rows, axis=0)

    gi = jnp.dot(pf, wih_ref[...], preferred_element_type=jnp.float32) + bih_ref[...]
    gh = _dot_t(h, whh_ref[...]) + bhh_ref[...]
    i_r, i_z, i_n = gi[:, :G], gi[:, G:2 * G], gi[:, 2 * G:]
    h_r, h_z, h_n = gh[:, :G], gh[:, G:2 * G], gh[:, 2 * G:]
    r = jax.nn.sigmoid(i_r + h_r)
    z = jax.nn.sigmoid(i_z + h_z)
    n = jnp.tanh(i_n + r * h_n)
    hn = (1.0 - z) * n + z * h

    x1 = jax.nn.relu(_dot_t(hn, l1w_ref[...]) + l1b_ref[...])
    x2 = jax.nn.relu(_dot_t(x1, l2aw_ref[...]) + l2ab_ref[...])
    x2 = jax.nn.relu(_dot_t(x2, l2bw_ref[...]) + l2bb_ref[...])

    gw = ln2g_ref[...] * sw_ref[...]
    c2 = jnp.sum(ln2b_ref[...] * sw_ref[...]) + sb_ref[0]
    mu = jnp.mean(x2, axis=1, keepdims=True)
    d = x2 - mu
    v = jnp.mean(d * d, axis=1, keepdims=True)
    sgw = jnp.sum(d * gw, axis=1, keepdims=True)
    scores = jax.lax.rsqrt(v + 1e-5) * sgw + c2

    iota32 = jax.lax.broadcasted_iota(jnp.int32, (32, S), 0).astype(jnp.float32)
    iota16 = jax.lax.broadcasted_iota(jnp.int32, (16, S), 0).astype(jnp.float32)
    for b in range(B):
        sc = scores[b * S:(b + 1) * S]
        smax = jnp.max(sc, axis=0, keepdims=True)
        e = jnp.exp(sc - smax)
        probs = e / jnp.sum(e, axis=0, keepdims=True)
        probs_row = jnp.transpose(probs, (1, 0))
        probs_ref[b] = probs_row[0]

        pk = peaks_ref[b]
        bins = jnp.clip(jnp.round(pk), 0.0, float(_SPECT_BIN_N - 1))
        f16 = jnp.floor(bins * (1.0 / 32.0))
        c32 = bins - 32.0 * f16
        contrib = pk * probs_row
        wc_parts = []
        f_parts = []
        for m in range(M2 // 2):
            ohc = (c32[2 * m:2 * m + 1] == iota32).astype(jnp.float32)
            wc_parts.append(contrib[2 * m + 1:2 * m + 2] * ohc)
            f_parts.append((f16[2 * m:2 * m + 1] == iota16).astype(jnp.float32))
        wc = jnp.concatenate(wc_parts, axis=1)
        fh = jnp.concatenate(f_parts, axis=1)
        out = jax.lax.dot_general(
            fh, wc, (((1,), (1,)), ((), ())),
            preferred_element_type=jnp.float32)
        for f in range(16):
            spect_ref[b, pl.ds(32 * f, 32)] = out[f]


def kernel(vert_feat_in, vert_mask_in, vert_element_oh, adj_oh, atom_subsets,
           atom_subsets_peaks, ln_g, ln_b, gru_w_ih, gru_w_hh, gru_b_ih,
           gru_b_hh, l1_w, l1_b, l2a_w, l2a_b, l2b_w, l2b_b, ln2_g, ln2_b,
           score_w, score_b):
    B, A, GF0, HW = vert_feat_in.shape
    G = GF0 * HW
    S = atom_subsets.shape[1]
    M = atom_subsets_peaks.shape[2]
    E = vert_element_oh.shape[2]

    x_v = vert_feat_in.transpose(0, 1, 3, 2).reshape(B, A, G)
    mask3 = vert_mask_in.reshape(B, 1, A)
    elem_v = vert_element_oh.transpose(2, 0, 1).reshape(E, B * A)
    subs_v = atom_subsets.transpose(0, 2, 1)
    peaks_v = atom_subsets_peaks.transpose(0, 2, 3, 1).reshape(B, 2 * M, S)
    wih_v = gru_w_ih.T

    spect, probs = pl.pallas_call(
        _full_kernel,
        out_shape=[
            jax.ShapeDtypeStruct((B, _SPECT_BIN_N), jnp.float32),
            jax.ShapeDtypeStruct((B, S), jnp.float32),
        ],
    )(
        x_v, mask3, elem_v, subs_v, peaks_v,
        ln_g, ln_b,
        wih_v, gru_w_hh, gru_b_ih, gru_b_hh,
        l1_w, l1_b, l2a_w, l2a_b, l2b_w, l2b_b,
        ln2_g, ln2_b, score_w, score_b,
    )
    return spect, probs


# speedup vs baseline: 2.0925x; 1.0000x over previous
import jax
import jax.numpy as jnp
import numpy as np
from jax.experimental import pallas as pl

_FORMULA_OH_SIZES = [20, 20, 20, 20, 20]
_SPECT_BIN_N = 512


def _dot_t(x, w):
    return jax.lax.dot_general(
        x, w, (((1,), (1,)), ((), ())), preferred_element_type=jnp.float32)


def _dot_tl(xt, y):
    return jax.lax.dot_general(
        xt, y, (((0,), (0,)), ((), ())), preferred_element_type=jnp.float32)


def _ln(x, g, b, eps=1e-5):
    m = jnp.mean(x, axis=-1, keepdims=True)
    v = jnp.mean((x - m) ** 2, axis=-1, keepdims=True)
    return (x - m) * jax.lax.rsqrt(v + eps) * g + b


def _full_kernel(
    x_ref,
    mask_ref,
    elem_ref,
    subs_ref,
    peaks_ref,
    ln_g_ref, ln_b_ref,
    wih_ref,
    whh_ref,
    bih_ref, bhh_ref,
    l1w_ref,
    l1b_ref,
    l2aw_ref, l2ab_ref,
    l2bw_ref, l2bb_ref,
    ln2g_ref, ln2b_ref,
    sw_ref,
    sb_ref,
    spect_ref,
    probs_ref,
):
    B, A, S = subs_ref.shape
    G = x_ref.shape[2]
    M2 = peaks_ref.shape[1]
    E = elem_ref.shape[0]
    F = 20 * E
    HW = 4
    GF0 = G // HW

    rowp = jax.lax.broadcasted_iota(jnp.int32, (G, G), 0)
    colp = jax.lax.broadcasted_iota(jnp.int32, (G, G), 1)
    perm = ((rowp % GF0) * HW + rowp // GF0 == colp).astype(jnp.float32)
    x_all = jnp.dot(x_ref[...].reshape(B * A, G), perm,
                    preferred_element_type=jnp.float32)

    col = jax.lax.broadcasted_iota(jnp.int32, (E, F), 1)
    rowi = jax.lax.broadcasted_iota(jnp.int32, (E, F), 0)
    seg = (col // 20 == rowi).astype(jnp.float32)
    p_all = _dot_tl(elem_ref[...].astype(jnp.float32), seg)
    colmod = (jax.lax.broadcasted_iota(jnp.int32, (S, F), 1) % 20).astype(jnp.float32)

    ones_a = jnp.ones((A, 1), jnp.float32)

    h_rows = []
    pf_rows = []
    for b in range(B):
        subs_t = subs_ref[b].astype(jnp.float32)
        mask_t = mask_ref[b].reshape(A, 1)
        subs_m = subs_t * mask_t
        subs_mm = subs_m * mask_t

        x_b = x_all[b * A:(b + 1) * A]
        swvs = _dot_tl(subs_mm, x_b)
        size = _dot_tl(subs_m, ones_a) + 0.0001
        h_rows.append(_ln(swvs / size, ln_g_ref[...], ln_b_ref[...]))

        cx = _dot_tl(subs_t, p_all[b * A:(b + 1) * A])
        thresh = jnp.clip(cx, 0.0, 19.0)
        pf_rows.append((colmod >= thresh).astype(jnp.float32))

    h = jnp.concatenate(h_rows, axis=0)
    pf = jnp.concatenate(pf_rows, axis=0)

    bf = jnp.bfloat16
    gi = jnp.dot(pf.astype(bf), wih_ref[...].astype(bf),
                 preferred_element_type=jnp.float32) + bih_ref[...]
    gh = _dot_t(h.astype(bf), whh_ref[...].astype(bf)) + bhh_ref[...]
    i_r, i_z, i_n = gi[:, :G], gi[:, G:2 * G], gi[:, 2 * G:]
    h_r, h_z, h_n = gh[:, :G], gh[:, G:2 * G], gh[:, 2 * G:]
    r = jax.nn.sigmoid(i_r + h_r)
    z = jax.nn.sigmoid(i_z + h_z)
    n = jnp.tanh(i_n + r * h_n)
    hn = (1.0 - z) * n + z * h

    x1 = jax.nn.relu(_dot_t(hn.astype(bf), l1w_ref[...].astype(bf)) + l1b_ref[...])
    x2 = jax.nn.relu(_dot_t(x1.astype(bf), l2aw_ref[...].astype(bf)) + l2ab_ref[...])
    x2 = jax.nn.relu(_dot_t(x2.astype(bf), l2bw_ref[...].astype(bf)) + l2bb_ref[...])

    gw = ln2g_ref[...] * sw_ref[...]
    c2 = jnp.sum(ln2b_ref[...] * sw_ref[...]) + sb_ref[0]
    mu = jnp.mean(x2, axis=1, keepdims=True)
    d = x2 - mu
    v = jnp.mean(d * d, axis=1, keepdims=True)
    sgw = jnp.sum(d * gw, axis=1, keepdims=True)
    scores = jax.lax.rsqrt(v + 1e-5) * sgw + c2

    iota32 = jax.lax.broadcasted_iota(jnp.int32, (32, S), 0).astype(jnp.float32)
    iota16 = jax.lax.broadcasted_iota(jnp.int32, (16, S), 0).astype(jnp.float32)
    for b in range(B):
        sc = scores[b * S:(b + 1) * S]
        smax = jnp.max(sc, axis=0, keepdims=True)
        e = jnp.exp(sc - smax)
        probs = e / jnp.sum(e, axis=0, keepdims=True)
        probs_row = jnp.transpose(probs, (1, 0))
        probs_ref[b] = probs_row[0]

        pk = peaks_ref[b]
        bins = jnp.clip(jnp.round(pk), 0.0, float(_SPECT_BIN_N - 1))
        f16 = jnp.floor(bins * (1.0 / 32.0))
        c32 = bins - 32.0 * f16
        contrib = pk * probs_row
        wc_parts = []
        f_parts = []
        for m in range(M2 // 2):
            ohc = (c32[2 * m:2 * m + 1] == iota32).astype(jnp.float32)
            wc_parts.append(contrib[2 * m + 1:2 * m + 2] * ohc)
            f_parts.append((f16[2 * m:2 * m + 1] == iota16).astype(jnp.float32))
        wc = jnp.concatenate(wc_parts, axis=1)
        fh = jnp.concatenate(f_parts, axis=1)
        out = jax.lax.dot_general(
            fh, wc, (((1,), (1,)), ((), ())),
            preferred_element_type=jnp.float32)
        for f in range(16):
            spect_ref[b, pl.ds(32 * f, 32)] = out[f]


def kernel(vert_feat_in, vert_mask_in, vert_element_oh, adj_oh, atom_subsets,
           atom_subsets_peaks, ln_g, ln_b, gru_w_ih, gru_w_hh, gru_b_ih,
           gru_b_hh, l1_w, l1_b, l2a_w, l2a_b, l2b_w, l2b_b, ln2_g, ln2_b,
           score_w, score_b):
    B, A, GF0, HW = vert_feat_in.shape
    G = GF0 * HW
    S = atom_subsets.shape[1]
    M = atom_subsets_peaks.shape[2]
    E = vert_element_oh.shape[2]

    x_v = vert_feat_in.transpose(0, 1, 3, 2).reshape(B, A, G)
    mask3 = vert_mask_in.reshape(B, 1, A)
    elem_v = vert_element_oh.transpose(2, 0, 1).reshape(E, B * A)
    subs_v = atom_subsets.transpose(0, 2, 1)
    peaks_v = atom_subsets_peaks.transpose(0, 2, 3, 1).reshape(B, 2 * M, S)
    wih_v = gru_w_ih.T

    spect, probs = pl.pallas_call(
        _full_kernel,
        out_shape=[
            jax.ShapeDtypeStruct((B, _SPECT_BIN_N), jnp.float32),
            jax.ShapeDtypeStruct((B, S), jnp.float32),
        ],
    )(
        x_v, mask3, elem_v, subs_v, peaks_v,
        ln_g, ln_b,
        wih_v, gru_w_hh, gru_b_ih, gru_b_hh,
        l1_w, l1_b, l2a_w, l2a_b, l2b_w, l2b_b,
        ln2_g, ln2_b, score_w, score_b,
    )
    return spect, probs
